# baseline (speedup 1.0000x reference)
import functools

import jax
import jax.numpy as jnp
from jax import lax
from jax.experimental import pallas as pl
from jax.experimental.pallas import tpu as pltpu
from jax.experimental.pallas import tpu_sc as plsc

N = 10000
E = 320000
D_IN = 128
H = 256
D_OUT = 3

NPAD = 10240
CHUNK = 128
NSUB = 16
NCORE = 2
C = 160
EPAD = NSUB * C * CHUNK
G = 32
NG = C // G
BM = 512
GRID = NPAD // BM
STRIPE = NPAD // NSUB

_mesh = plsc.VectorSubcoreMesh(core_axis_name="c", subcore_axis_name="s")


@functools.partial(
    pl.kernel,
    out_type=jax.ShapeDtypeStruct((NCORE * NPAD, 128), jnp.float32),
    mesh=_mesh,
    scratch_types=[
        pltpu.VMEM((C, CHUNK), jnp.int32),
        pltpu.VMEM((CHUNK, 128), jnp.float32),
        pltpu.VMEM_SHARED((NPAD, 128), jnp.float32),
    ],
)
def _deg_kernel(dst_hbm, ones_hbm, zeros_hbm, out_hbm, idx_v, ones_v, deg_sh):
    c = lax.axis_index("c")
    s = lax.axis_index("s")
    pltpu.sync_copy(dst_hbm.at[s], idx_v)
    pltpu.sync_copy(ones_hbm, ones_v)
    pltpu.sync_copy(zeros_hbm.at[pl.ds(s * STRIPE, STRIPE)],
                    deg_sh.at[pl.ds(s * STRIPE, STRIPE)])
    plsc.subcore_barrier()
    lo = c * (C // 2)

    def body(j, _):
        pltpu.sync_copy(ones_v, deg_sh.at[idx_v.at[j]], add=True)
        return 0

    lax.fori_loop(lo, lo + C // 2, body, 0)
    plsc.subcore_barrier()
    pltpu.sync_copy(deg_sh.at[pl.ds(s * STRIPE, STRIPE)],
                    out_hbm.at[pl.ds(c * NPAD + s * STRIPE, STRIPE)])


@functools.partial(
    pl.kernel,
    out_type=jax.ShapeDtypeStruct((NCORE * NPAD, 128), jnp.float32),
    mesh=_mesh,
    scratch_types=[
        pltpu.VMEM((G, CHUNK), jnp.int32),
        pltpu.VMEM((G, CHUNK), jnp.int32),
        pltpu.VMEM((CHUNK, 128), jnp.float32),
        pltpu.VMEM((CHUNK, 128), jnp.float32),
        pltpu.VMEM_SHARED((NPAD, 128), jnp.float32),
        pltpu.SemaphoreType.DMA,
        pltpu.SemaphoreType.DMA,
        pltpu.SemaphoreType.DMA,
        pltpu.SemaphoreType.DMA,
    ],
)
def _agg_kernel(plo_hbm, phi_hbm, src_hbm, dst_hbm, zeros_hbm, out_hbm,
                src_v, dst_v, buf_a, buf_b, acc_sh, sem_a, sem_b, sem_sa,
                sem_sb):
    c = lax.axis_index("c")
    s = lax.axis_index("s")
    pltpu.sync_copy(zeros_hbm.at[pl.ds(s * STRIPE, STRIPE)],
                    acc_sh.at[pl.ds(s * STRIPE, STRIPE)])
    plsc.subcore_barrier()

    def run(tbl):
        def outer(g, _):
            pltpu.sync_copy(src_hbm.at[s, pl.ds(g * G, G)], src_v)
            pltpu.sync_copy(dst_hbm.at[s, pl.ds(g * G, G)], dst_v)
            pltpu.async_copy(tbl.at[src_v.at[0]], buf_a, sem_a)
            pltpu.async_copy(tbl.at[src_v.at[1]], buf_b, sem_b)

            def body(jj, _):
                j0 = 2 * jj
                pltpu.make_async_copy(tbl.at[src_v.at[j0]], buf_a,
                                      sem_a).wait()
                pltpu.async_copy(buf_a, acc_sh.at[dst_v.at[j0]], sem_sa,
                                 add=True)
                pltpu.make_async_copy(tbl.at[src_v.at[j0 + 1]], buf_b,
                                      sem_b).wait()
                pltpu.async_copy(buf_b, acc_sh.at[dst_v.at[j0 + 1]], sem_sb,
                                 add=True)
                j2 = jnp.minimum(j0 + 2, G - 1)
                j3 = jnp.minimum(j0 + 3, G - 1)
                pltpu.make_async_copy(buf_a, acc_sh.at[dst_v.at[j0]],
                                      sem_sa).wait()
                pltpu.async_copy(tbl.at[src_v.at[j2]], buf_a, sem_a)
                pltpu.make_async_copy(buf_b, acc_sh.at[dst_v.at[j0 + 1]],
                                      sem_sb).wait()
                pltpu.async_copy(tbl.at[src_v.at[j3]], buf_b, sem_b)
                return 0

            lax.fori_loop(0, G // 2, body, 0)
            pltpu.make_async_copy(tbl.at[src_v.at[0]], buf_a, sem_a).wait()
            pltpu.make_async_copy(tbl.at[src_v.at[0]], buf_b, sem_b).wait()
            return 0

        lax.fori_loop(0, NG, outer, 0)

    @pl.when(c == 0)
    def _():
        run(plo_hbm)

    @pl.when(c == 1)
    def _():
        run(phi_hbm)

    plsc.subcore_barrier()
    pltpu.sync_copy(acc_sh.at[pl.ds(s * STRIPE, STRIPE)],
                    out_hbm.at[pl.ds(c * NPAD + s * STRIPE, STRIPE)])


def _dense1_body(x_ref, w1_ref, dega_ref, degb_ref, plo_ref, phi_ref, dis_ref):
    i = pl.program_id(0)
    h = jnp.dot(x_ref[...], w1_ref[...], preferred_element_type=jnp.float32)
    deg = dega_ref[...][:, :1] + degb_ref[...][:, :1] + 1.0
    rid = lax.broadcasted_iota(jnp.int32, (BM, 1), 0) + i * BM
    dis = jnp.where(rid < N, lax.rsqrt(deg), 0.0)
    p = h * dis
    plo_ref[...] = p[:, :128]
    phi_ref[...] = p[:, 128:]
    dis_ref[...] = dis


def _dense1(x_pad, w1, deg_parts):
    return pl.pallas_call(
        _dense1_body,
        grid=(GRID,),
        in_specs=[
            pl.BlockSpec((BM, D_IN), lambda i: (i, 0)),
            pl.BlockSpec((D_IN, H), lambda i: (0, 0)),
            pl.BlockSpec((BM, 128), lambda i: (i, 0)),
            pl.BlockSpec((BM, 128), lambda i: (i + GRID, 0)),
        ],
        out_specs=[
            pl.BlockSpec((BM, 128), lambda i: (i, 0)),
            pl.BlockSpec((BM, 128), lambda i: (i, 0)),
            pl.BlockSpec((BM, 1), lambda i: (i, 0)),
        ],
        out_shape=[
            jax.ShapeDtypeStruct((NPAD, 128), jnp.float32),
            jax.ShapeDtypeStruct((NPAD, 128), jnp.float32),
            jax.ShapeDtypeStruct((NPAD, 1), jnp.float32),
        ],
    )(x_pad, w1, deg_parts, deg_parts)


def _dense2_body(alo_ref, ahi_ref, plo_ref, phi_ref, dis_ref, b1_ref, w2_ref,
                 p2lo_ref, p2hi_ref):
    dis = dis_ref[...]
    agg = jnp.concatenate(
        [alo_ref[...] + plo_ref[...], ahi_ref[...] + phi_ref[...]], axis=1)
    o = jnp.maximum(agg * dis + b1_ref[...], 0.0)
    h2 = jnp.dot(o, w2_ref[...], preferred_element_type=jnp.float32)
    p2 = h2 * dis
    p2lo_ref[...] = p2[:, :128]
    p2hi_ref[...] = p2[:, 128:]


def _dense2(agg, plo, phi, dis, b1, w2):
    return pl.pallas_call(
        _dense2_body,
        grid=(GRID,),
        in_specs=[
            pl.BlockSpec((BM, 128), lambda i: (i, 0)),
            pl.BlockSpec((BM, 128), lambda i: (i + GRID, 0)),
            pl.BlockSpec((BM, 128), lambda i: (i, 0)),
            pl.BlockSpec((BM, 128), lambda i: (i, 0)),
            pl.BlockSpec((BM, 1), lambda i: (i, 0)),
            pl.BlockSpec((1, H), lambda i: (0, 0)),
            pl.BlockSpec((H, H), lambda i: (0, 0)),
        ],
        out_specs=[
            pl.BlockSpec((BM, 128), lambda i: (i, 0)),
            pl.BlockSpec((BM, 128), lambda i: (i, 0)),
        ],
        out_shape=[
            jax.ShapeDtypeStruct((NPAD, 128), jnp.float32),
            jax.ShapeDtypeStruct((NPAD, 128), jnp.float32),
        ],
    )(agg, agg, plo, phi, dis, b1, w2)


def _dense3_body(alo_ref, ahi_ref, plo_ref, phi_ref, dis_ref, b2_ref,
                 wfc_ref, bfc_ref, out_ref):
    dis = dis_ref[...]
    agg = jnp.concatenate(
        [alo_ref[...] + plo_ref[...], ahi_ref[...] + phi_ref[...]], axis=1)
    o = jnp.maximum(agg * dis + b2_ref[...], 0.0)
    l = jnp.dot(o, wfc_ref[...], preferred_element_type=jnp.float32)
    l = l + bfc_ref[...]
    col = lax.broadcasted_iota(jnp.int32, (BM, 128), 1)
    valid = col < D_OUT
    m = jnp.max(jnp.where(valid, l, -1e30), axis=1, keepdims=True)
    e = jnp.where(valid, jnp.exp(l - m), 0.0)
    lse = m + jnp.log(jnp.sum(e, axis=1, keepdims=True))
    out_ref[...] = l - lse


def _dense3(agg, plo, phi, dis, b2, wfc_pad, bfc_pad):
    return pl.pallas_call(
        _dense3_body,
        grid=(GRID,),
        in_specs=[
            pl.BlockSpec((BM, 128), lambda i: (i, 0)),
            pl.BlockSpec((BM, 128), lambda i: (i + GRID, 0)),
            pl.BlockSpec((BM, 128), lambda i: (i, 0)),
            pl.BlockSpec((BM, 128), lambda i: (i, 0)),
            pl.BlockSpec((BM, 1), lambda i: (i, 0)),
            pl.BlockSpec((1, H), lambda i: (0, 0)),
            pl.BlockSpec((H, 128), lambda i: (0, 0)),
            pl.BlockSpec((1, 128), lambda i: (0, 0)),
        ],
        out_specs=pl.BlockSpec((BM, 128), lambda i: (i, 0)),
        out_shape=jax.ShapeDtypeStruct((NPAD, 128), jnp.float32),
    )(agg, agg, plo, phi, dis, b2, wfc_pad, bfc_pad)


def kernel(x, edge_index, W1, b1, W2, b2, Wfc, bfc):
    src = edge_index[0].astype(jnp.int32)
    dst = edge_index[1].astype(jnp.int32)
    fill = jnp.full((EPAD - E,), N, jnp.int32)
    src_g = jnp.concatenate([src, fill]).reshape(NSUB, C, CHUNK)
    dst_g = jnp.concatenate([dst, fill]).reshape(NSUB, C, CHUNK)
    x_pad = jnp.pad(x, ((0, NPAD - N), (0, 0)))
    ones128 = jnp.ones((CHUNK, 128), jnp.float32)
    z_big = jnp.zeros((NPAD, 128), jnp.float32)

    deg_parts = _deg_kernel(dst_g, ones128, z_big)
    plo, phi, dis = _dense1(x_pad, W1, deg_parts)
    agg1 = _agg_kernel(plo, phi, src_g, dst_g, z_big)
    p2lo, p2hi = _dense2(agg1, plo, phi, dis, b1.reshape(1, H), W2)
    agg2 = _agg_kernel(p2lo, p2hi, src_g, dst_g, z_big)
    wfc_pad = jnp.pad(Wfc, ((0, 0), (0, 128 - D_OUT)))
    bfc_pad = jnp.pad(bfc, (0, 128 - D_OUT)).reshape(1, 128)
    outp = _dense3(agg2, p2lo, p2hi, dis, b2.reshape(1, H), wfc_pad, bfc_pad)
    return outp[:N, :D_OUT]


# speedup vs baseline: 1.0779x; 1.0779x over previous
import functools

import jax
import jax.numpy as jnp
from jax import lax
from jax.experimental import pallas as pl
from jax.experimental.pallas import tpu as pltpu
from jax.experimental.pallas import tpu_sc as plsc

N = 10000
E = 320000
D_IN = 128
H = 256
D_OUT = 3

NPAD = 10240
CHUNK = 128
NSUB = 16
NCORE = 2
C = 160
EPAD = NSUB * C * CHUNK
G = 40
NG = C // G
BM = 512
GRID = NPAD // BM
STRIPE = NPAD // NSUB

_mesh = plsc.VectorSubcoreMesh(core_axis_name="c", subcore_axis_name="s")


@functools.partial(
    pl.kernel,
    out_type=jax.ShapeDtypeStruct((NCORE * NPAD, 128), jnp.float32),
    mesh=_mesh,
    scratch_types=[
        pltpu.VMEM((C, CHUNK), jnp.int32),
        pltpu.VMEM((CHUNK, 128), jnp.float32),
        pltpu.VMEM_SHARED((NPAD, 128), jnp.float32),
    ],
)
def _deg_kernel(dst_hbm, ones_hbm, zeros_hbm, out_hbm, idx_v, ones_v, deg_sh):
    c = lax.axis_index("c")
    s = lax.axis_index("s")
    pltpu.sync_copy(dst_hbm.at[s], idx_v)
    pltpu.sync_copy(ones_hbm, ones_v)
    pltpu.sync_copy(zeros_hbm.at[pl.ds(s * STRIPE, STRIPE)],
                    deg_sh.at[pl.ds(s * STRIPE, STRIPE)])
    plsc.subcore_barrier()
    lo = c * (C // 2)

    def body(j, _):
        pltpu.sync_copy(ones_v, deg_sh.at[idx_v.at[j]], add=True)
        return 0

    lax.fori_loop(lo, lo + C // 2, body, 0)
    plsc.subcore_barrier()
    pltpu.sync_copy(deg_sh.at[pl.ds(s * STRIPE, STRIPE)],
                    out_hbm.at[pl.ds(c * NPAD + s * STRIPE, STRIPE)])


@functools.partial(
    pl.kernel,
    out_type=jax.ShapeDtypeStruct((NCORE * NPAD, 128), jnp.float32),
    mesh=_mesh,
    scratch_types=[
        pltpu.VMEM((G, CHUNK), jnp.int32),
        pltpu.VMEM((G, CHUNK), jnp.int32),
        pltpu.VMEM((CHUNK, 128), jnp.float32),
        pltpu.VMEM((CHUNK, 128), jnp.float32),
        pltpu.VMEM_SHARED((NPAD, 128), jnp.float32),
        pltpu.SemaphoreType.DMA,
        pltpu.SemaphoreType.DMA,
    ],
)
def _agg_kernel(plo_hbm, phi_hbm, src_hbm, dst_hbm, zeros_hbm, out_hbm,
                src_v, dst_v, buf_a, buf_b, acc_sh, sem_a, sem_b):
    c = lax.axis_index("c")
    s = lax.axis_index("s")
    pltpu.sync_copy(zeros_hbm.at[pl.ds(s * STRIPE, STRIPE)],
                    acc_sh.at[pl.ds(s * STRIPE, STRIPE)])
    plsc.subcore_barrier()

    def run(tbl):
        def outer(g, _):
            pltpu.sync_copy(src_hbm.at[s, pl.ds(g * G, G)], src_v)
            pltpu.sync_copy(dst_hbm.at[s, pl.ds(g * G, G)], dst_v)
            pltpu.async_copy(tbl.at[src_v.at[0]], buf_a, sem_a)
            pltpu.async_copy(tbl.at[src_v.at[1]], buf_b, sem_b)

            def body(jj, _):
                j0 = 2 * jj
                pltpu.make_async_copy(tbl.at[src_v.at[j0]], buf_a,
                                      sem_a).wait()
                pltpu.sync_copy(buf_a, acc_sh.at[dst_v.at[j0]], add=True)
                j2 = jnp.minimum(j0 + 2, G - 1)
                pltpu.async_copy(tbl.at[src_v.at[j2]], buf_a, sem_a)
                pltpu.make_async_copy(tbl.at[src_v.at[j0 + 1]], buf_b,
                                      sem_b).wait()
                pltpu.sync_copy(buf_b, acc_sh.at[dst_v.at[j0 + 1]], add=True)
                j3 = jnp.minimum(j0 + 3, G - 1)
                pltpu.async_copy(tbl.at[src_v.at[j3]], buf_b, sem_b)
                return 0

            lax.fori_loop(0, G // 2, body, 0)
            pltpu.make_async_copy(tbl.at[src_v.at[0]], buf_a, sem_a).wait()
            pltpu.make_async_copy(tbl.at[src_v.at[0]], buf_b, sem_b).wait()
            return 0

        lax.fori_loop(0, NG, outer, 0)

    @pl.when(c == 0)
    def _():
        run(plo_hbm)

    @pl.when(c == 1)
    def _():
        run(phi_hbm)

    plsc.subcore_barrier()
    pltpu.sync_copy(acc_sh.at[pl.ds(s * STRIPE, STRIPE)],
                    out_hbm.at[pl.ds(c * NPAD + s * STRIPE, STRIPE)])


def _dense1_body(x_ref, w1_ref, dega_ref, degb_ref, plo_ref, phi_ref, dis_ref):
    i = pl.program_id(0)
    h = jnp.dot(x_ref[...], w1_ref[...], preferred_element_type=jnp.float32)
    deg = dega_ref[...][:, :1] + degb_ref[...][:, :1] + 1.0
    rid = lax.broadcasted_iota(jnp.int32, (BM, 1), 0) + i * BM
    dis = jnp.where(rid < N, lax.rsqrt(deg), 0.0)
    p = h * dis
    plo_ref[...] = p[:, :128]
    phi_ref[...] = p[:, 128:]
    dis_ref[...] = dis


def _dense1(x_pad, w1, deg_parts):
    return pl.pallas_call(
        _dense1_body,
        grid=(GRID,),
        in_specs=[
            pl.BlockSpec((BM, D_IN), lambda i: (i, 0)),
            pl.BlockSpec((D_IN, H), lambda i: (0, 0)),
            pl.BlockSpec((BM, 128), lambda i: (i, 0)),
            pl.BlockSpec((BM, 128), lambda i: (i + GRID, 0)),
        ],
        out_specs=[
            pl.BlockSpec((BM, 128), lambda i: (i, 0)),
            pl.BlockSpec((BM, 128), lambda i: (i, 0)),
            pl.BlockSpec((BM, 1), lambda i: (i, 0)),
        ],
        out_shape=[
            jax.ShapeDtypeStruct((NPAD, 128), jnp.float32),
            jax.ShapeDtypeStruct((NPAD, 128), jnp.float32),
            jax.ShapeDtypeStruct((NPAD, 1), jnp.float32),
        ],
    )(x_pad, w1, deg_parts, deg_parts)


def _dense2_body(alo_ref, ahi_ref, plo_ref, phi_ref, dis_ref, b1_ref, w2_ref,
                 p2lo_ref, p2hi_ref):
    dis = dis_ref[...]
    agg = jnp.concatenate(
        [alo_ref[...] + plo_ref[...], ahi_ref[...] + phi_ref[...]], axis=1)
    o = jnp.maximum(agg * dis + b1_ref[...], 0.0)
    h2 = jnp.dot(o, w2_ref[...], preferred_element_type=jnp.float32)
    p2 = h2 * dis
    p2lo_ref[...] = p2[:, :128]
    p2hi_ref[...] = p2[:, 128:]


def _dense2(agg, plo, phi, dis, b1, w2):
    return pl.pallas_call(
        _dense2_body,
        grid=(GRID,),
        in_specs=[
            pl.BlockSpec((BM, 128), lambda i: (i, 0)),
            pl.BlockSpec((BM, 128), lambda i: (i + GRID, 0)),
            pl.BlockSpec((BM, 128), lambda i: (i, 0)),
            pl.BlockSpec((BM, 128), lambda i: (i, 0)),
            pl.BlockSpec((BM, 1), lambda i: (i, 0)),
            pl.BlockSpec((1, H), lambda i: (0, 0)),
            pl.BlockSpec((H, H), lambda i: (0, 0)),
        ],
        out_specs=[
            pl.BlockSpec((BM, 128), lambda i: (i, 0)),
            pl.BlockSpec((BM, 128), lambda i: (i, 0)),
        ],
        out_shape=[
            jax.ShapeDtypeStruct((NPAD, 128), jnp.float32),
            jax.ShapeDtypeStruct((NPAD, 128), jnp.float32),
        ],
    )(agg, agg, plo, phi, dis, b1, w2)


def _dense3_body(alo_ref, ahi_ref, plo_ref, phi_ref, dis_ref, b2_ref,
                 wfc_ref, bfc_ref, out_ref):
    dis = dis_ref[...]
    agg = jnp.concatenate(
        [alo_ref[...] + plo_ref[...], ahi_ref[...] + phi_ref[...]], axis=1)
    o = jnp.maximum(agg * dis + b2_ref[...], 0.0)
    l = jnp.dot(o, wfc_ref[...], preferred_element_type=jnp.float32)
    l = l + bfc_ref[...]
    col = lax.broadcasted_iota(jnp.int32, (BM, 128), 1)
    valid = col < D_OUT
    m = jnp.max(jnp.where(valid, l, -1e30), axis=1, keepdims=True)
    e = jnp.where(valid, jnp.exp(l - m), 0.0)
    lse = m + jnp.log(jnp.sum(e, axis=1, keepdims=True))
    out_ref[...] = l - lse


def _dense3(agg, plo, phi, dis, b2, wfc_pad, bfc_pad):
    return pl.pallas_call(
        _dense3_body,
        grid=(GRID,),
        in_specs=[
            pl.BlockSpec((BM, 128), lambda i: (i, 0)),
            pl.BlockSpec((BM, 128), lambda i: (i + GRID, 0)),
            pl.BlockSpec((BM, 128), lambda i: (i, 0)),
            pl.BlockSpec((BM, 128), lambda i: (i, 0)),
            pl.BlockSpec((BM, 1), lambda i: (i, 0)),
            pl.BlockSpec((1, H), lambda i: (0, 0)),
            pl.BlockSpec((H, 128), lambda i: (0, 0)),
            pl.BlockSpec((1, 128), lambda i: (0, 0)),
        ],
        out_specs=pl.BlockSpec((BM, 128), lambda i: (i, 0)),
        out_shape=jax.ShapeDtypeStruct((NPAD, 128), jnp.float32),
    )(agg, agg, plo, phi, dis, b2, wfc_pad, bfc_pad)


def kernel(x, edge_index, W1, b1, W2, b2, Wfc, bfc):
    src = edge_index[0].astype(jnp.int32)
    dst = edge_index[1].astype(jnp.int32)
    fill = jnp.full((EPAD - E,), N, jnp.int32)
    src_g = jnp.concatenate([src, fill]).reshape(NSUB, C, CHUNK)
    dst_g = jnp.concatenate([dst, fill]).reshape(NSUB, C, CHUNK)
    x_pad = jnp.pad(x, ((0, NPAD - N), (0, 0)))
    ones128 = jnp.ones((CHUNK, 128), jnp.float32)
    z_big = jnp.zeros((NPAD, 128), jnp.float32)

    deg_parts = _deg_kernel(dst_g, ones128, z_big)
    plo, phi, dis = _dense1(x_pad, W1, deg_parts)
    agg1 = _agg_kernel(plo, phi, src_g, dst_g, z_big)
    p2lo, p2hi = _dense2(agg1, plo, phi, dis, b1.reshape(1, H), W2)
    agg2 = _agg_kernel(p2lo, p2hi, src_g, dst_g, z_big)
    wfc_pad = jnp.pad(Wfc, ((0, 0), (0, 128 - D_OUT)))
    bfc_pad = jnp.pad(bfc, (0, 128 - D_OUT)).reshape(1, 128)
    outp = _dense3(agg2, p2lo, p2hi, dis, b2.reshape(1, H), wfc_pad, bfc_pad)
    return outp[:N, :D_OUT]


# speedup vs baseline: 1.1017x; 1.0220x over previous
import functools

import jax
import jax.numpy as jnp
from jax import lax
from jax.experimental import pallas as pl
from jax.experimental.pallas import tpu as pltpu
from jax.experimental.pallas import tpu_sc as plsc

N = 10000
E = 320000
D_IN = 128
H = 256
D_OUT = 3

NPAD = 10240
CHUNK = 128
NSUB = 16
NCORE = 2
C = 160
EPAD = NSUB * C * CHUNK
G = 32
NG = C // G
BM = 512
GRID = NPAD // BM
STRIPE = NPAD // NSUB

_mesh = plsc.VectorSubcoreMesh(core_axis_name="c", subcore_axis_name="s")


@functools.partial(
    pl.kernel,
    out_type=jax.ShapeDtypeStruct((NCORE * NPAD, 128), jnp.float32),
    mesh=_mesh,
    scratch_types=[
        pltpu.VMEM((C, CHUNK), jnp.int32),
        pltpu.VMEM((CHUNK, 128), jnp.float32),
        pltpu.VMEM_SHARED((NPAD, 128), jnp.float32),
    ],
)
def _deg_kernel(dst_hbm, ones_hbm, zeros_hbm, out_hbm, idx_v, ones_v, deg_sh):
    c = lax.axis_index("c")
    s = lax.axis_index("s")
    pltpu.sync_copy(dst_hbm.at[s], idx_v)
    pltpu.sync_copy(ones_hbm, ones_v)
    pltpu.sync_copy(zeros_hbm.at[pl.ds(s * STRIPE, STRIPE)],
                    deg_sh.at[pl.ds(s * STRIPE, STRIPE)])
    plsc.subcore_barrier()
    lo = c * (C // 2)

    def body(j, _):
        pltpu.sync_copy(ones_v, deg_sh.at[idx_v.at[j]], add=True)
        return 0

    lax.fori_loop(lo, lo + C // 2, body, 0)
    plsc.subcore_barrier()
    pltpu.sync_copy(deg_sh.at[pl.ds(s * STRIPE, STRIPE)],
                    out_hbm.at[pl.ds(c * NPAD + s * STRIPE, STRIPE)])


@functools.partial(
    pl.kernel,
    out_type=jax.ShapeDtypeStruct((NCORE * NPAD, 128), jnp.float32),
    mesh=_mesh,
    scratch_types=[
        pltpu.VMEM((G, CHUNK), jnp.int32),
        pltpu.VMEM((G, CHUNK), jnp.int32),
        pltpu.VMEM((CHUNK, 128), jnp.float32),
        pltpu.VMEM((CHUNK, 128), jnp.float32),
        pltpu.VMEM_SHARED((NPAD, 128), jnp.float32),
        pltpu.SemaphoreType.DMA,
        pltpu.SemaphoreType.DMA,
    ],
)
def _agg_kernel(plo_hbm, phi_hbm, src_hbm, dst_hbm, zeros_hbm, out_hbm,
                src_v, dst_v, buf_a, buf_b, acc_sh, sem_a, sem_b):
    c = lax.axis_index("c")
    s = lax.axis_index("s")
    pltpu.sync_copy(zeros_hbm.at[pl.ds(s * STRIPE, STRIPE)],
                    acc_sh.at[pl.ds(s * STRIPE, STRIPE)])
    plsc.subcore_barrier()

    def run(tbl):
        def outer(g, _):
            pltpu.sync_copy(src_hbm.at[s, pl.ds(g * G, G)], src_v)
            pltpu.sync_copy(dst_hbm.at[s, pl.ds(g * G, G)], dst_v)
            pltpu.async_copy(tbl.at[src_v.at[0]], buf_a, sem_a)

            def body(jj, _):
                j0 = 2 * jj
                pltpu.async_copy(tbl.at[src_v.at[j0 + 1]], buf_b, sem_b)
                pltpu.make_async_copy(tbl.at[src_v.at[j0]], buf_a,
                                      sem_a).wait()
                pltpu.sync_copy(buf_a, acc_sh.at[dst_v.at[j0]], add=True)
                j2 = jnp.minimum(j0 + 2, G - 1)
                pltpu.async_copy(tbl.at[src_v.at[j2]], buf_a, sem_a)
                pltpu.make_async_copy(tbl.at[src_v.at[j0 + 1]], buf_b,
                                      sem_b).wait()
                pltpu.sync_copy(buf_b, acc_sh.at[dst_v.at[j0 + 1]], add=True)
                return 0

            lax.fori_loop(0, G // 2, body, 0)
            pltpu.make_async_copy(tbl.at[src_v.at[0]], buf_a, sem_a).wait()
            return 0

        lax.fori_loop(0, NG, outer, 0)

    @pl.when(c == 0)
    def _():
        run(plo_hbm)

    @pl.when(c == 1)
    def _():
        run(phi_hbm)

    plsc.subcore_barrier()
    pltpu.sync_copy(acc_sh.at[pl.ds(s * STRIPE, STRIPE)],
                    out_hbm.at[pl.ds(c * NPAD + s * STRIPE, STRIPE)])


def _dense1_body(x_ref, w1_ref, dega_ref, degb_ref, plo_ref, phi_ref, dis_ref):
    i = pl.program_id(0)
    h = jnp.dot(x_ref[...], w1_ref[...], preferred_element_type=jnp.float32)
    deg = dega_ref[...][:, :1] + degb_ref[...][:, :1] + 1.0
    rid = lax.broadcasted_iota(jnp.int32, (BM, 1), 0) + i * BM
    dis = jnp.where(rid < N, lax.rsqrt(deg), 0.0)
    p = h * dis
    plo_ref[...] = p[:, :128]
    phi_ref[...] = p[:, 128:]
    dis_ref[...] = dis


def _dense1(x_pad, w1, deg_parts):
    return pl.pallas_call(
        _dense1_body,
        grid=(GRID,),
        in_specs=[
            pl.BlockSpec((BM, D_IN), lambda i: (i, 0)),
            pl.BlockSpec((D_IN, H), lambda i: (0, 0)),
            pl.BlockSpec((BM, 128), lambda i: (i, 0)),
            pl.BlockSpec((BM, 128), lambda i: (i + GRID, 0)),
        ],
        out_specs=[
            pl.BlockSpec((BM, 128), lambda i: (i, 0)),
            pl.BlockSpec((BM, 128), lambda i: (i, 0)),
            pl.BlockSpec((BM, 1), lambda i: (i, 0)),
        ],
        out_shape=[
            jax.ShapeDtypeStruct((NPAD, 128), jnp.float32),
            jax.ShapeDtypeStruct((NPAD, 128), jnp.float32),
            jax.ShapeDtypeStruct((NPAD, 1), jnp.float32),
        ],
    )(x_pad, w1, deg_parts, deg_parts)


def _dense2_body(alo_ref, ahi_ref, plo_ref, phi_ref, dis_ref, b1_ref, w2_ref,
                 p2lo_ref, p2hi_ref):
    dis = dis_ref[...]
    agg = jnp.concatenate(
        [alo_ref[...] + plo_ref[...], ahi_ref[...] + phi_ref[...]], axis=1)
    o = jnp.maximum(agg * dis + b1_ref[...], 0.0)
    h2 = jnp.dot(o, w2_ref[...], preferred_element_type=jnp.float32)
    p2 = h2 * dis
    p2lo_ref[...] = p2[:, :128]
    p2hi_ref[...] = p2[:, 128:]


def _dense2(agg, plo, phi, dis, b1, w2):
    return pl.pallas_call(
        _dense2_body,
        grid=(GRID,),
        in_specs=[
            pl.BlockSpec((BM, 128), lambda i: (i, 0)),
            pl.BlockSpec((BM, 128), lambda i: (i + GRID, 0)),
            pl.BlockSpec((BM, 128), lambda i: (i, 0)),
            pl.BlockSpec((BM, 128), lambda i: (i, 0)),
            pl.BlockSpec((BM, 1), lambda i: (i, 0)),
            pl.BlockSpec((1, H), lambda i: (0, 0)),
            pl.BlockSpec((H, H), lambda i: (0, 0)),
        ],
        out_specs=[
            pl.BlockSpec((BM, 128), lambda i: (i, 0)),
            pl.BlockSpec((BM, 128), lambda i: (i, 0)),
        ],
        out_shape=[
            jax.ShapeDtypeStruct((NPAD, 128), jnp.float32),
            jax.ShapeDtypeStruct((NPAD, 128), jnp.float32),
        ],
    )(agg, agg, plo, phi, dis, b1, w2)


def _dense3_body(alo_ref, ahi_ref, plo_ref, phi_ref, dis_ref, b2_ref,
                 wfc_ref, bfc_ref, out_ref):
    dis = dis_ref[...]
    agg = jnp.concatenate(
        [alo_ref[...] + plo_ref[...], ahi_ref[...] + phi_ref[...]], axis=1)
    o = jnp.maximum(agg * dis + b2_ref[...], 0.0)
    l = jnp.dot(o, wfc_ref[...], preferred_element_type=jnp.float32)
    l = l + bfc_ref[...]
    col = lax.broadcasted_iota(jnp.int32, (BM, 128), 1)
    valid = col < D_OUT
    m = jnp.max(jnp.where(valid, l, -1e30), axis=1, keepdims=True)
    e = jnp.where(valid, jnp.exp(l - m), 0.0)
    lse = m + jnp.log(jnp.sum(e, axis=1, keepdims=True))
    out_ref[...] = l - lse


def _dense3(agg, plo, phi, dis, b2, wfc_pad, bfc_pad):
    return pl.pallas_call(
        _dense3_body,
        grid=(GRID,),
        in_specs=[
            pl.BlockSpec((BM, 128), lambda i: (i, 0)),
            pl.BlockSpec((BM, 128), lambda i: (i + GRID, 0)),
            pl.BlockSpec((BM, 128), lambda i: (i, 0)),
            pl.BlockSpec((BM, 128), lambda i: (i, 0)),
            pl.BlockSpec((BM, 1), lambda i: (i, 0)),
            pl.BlockSpec((1, H), lambda i: (0, 0)),
            pl.BlockSpec((H, 128), lambda i: (0, 0)),
            pl.BlockSpec((1, 128), lambda i: (0, 0)),
        ],
        out_specs=pl.BlockSpec((BM, 128), lambda i: (i, 0)),
        out_shape=jax.ShapeDtypeStruct((NPAD, 128), jnp.float32),
    )(agg, agg, plo, phi, dis, b2, wfc_pad, bfc_pad)


def kernel(x, edge_index, W1, b1, W2, b2, Wfc, bfc):
    src = edge_index[0].astype(jnp.int32)
    dst = edge_index[1].astype(jnp.int32)
    fill = jnp.full((EPAD - E,), N, jnp.int32)
    src_g = jnp.concatenate([src, fill]).reshape(NSUB, C, CHUNK)
    dst_g = jnp.concatenate([dst, fill]).reshape(NSUB, C, CHUNK)
    x_pad = jnp.pad(x, ((0, NPAD - N), (0, 0)))
    ones128 = jnp.ones((CHUNK, 128), jnp.float32)
    z_big = jnp.zeros((NPAD, 128), jnp.float32)

    deg_parts = _deg_kernel(dst_g, ones128, z_big)
    plo, phi, dis = _dense1(x_pad, W1, deg_parts)
    agg1 = _agg_kernel(plo, phi, src_g, dst_g, z_big)
    p2lo, p2hi = _dense2(agg1, plo, phi, dis, b1.reshape(1, H), W2)
    agg2 = _agg_kernel(p2lo, p2hi, src_g, dst_g, z_big)
    wfc_pad = jnp.pad(Wfc, ((0, 0), (0, 128 - D_OUT)))
    bfc_pad = jnp.pad(bfc, (0, 128 - D_OUT)).reshape(1, 128)
    outp = _dense3(agg2, p2lo, p2hi, dis, b2.reshape(1, H), wfc_pad, bfc_pad)
    return outp[:N, :D_OUT]
